# baseline (speedup 1.0000x reference)
import functools

import jax
import jax.numpy as jnp
from jax import lax
from jax.experimental import pallas as pl
from jax.experimental.pallas import tpu as pltpu
from jax.experimental.pallas import tpu_sc as plsc

N_NODES = 10000
N_EDGES = 320000
D_IN = 128
D_HALF = 64
SPH_DIM = 9
D_EA = 4
EA_PAD = 16
K_BLOOM = 4
B_GRAPHS = 8

NC = 2
NS = 16
NW = NC * NS
G = 128
EPT = 10240
NGRP = EPT // G
EPT16 = 2 * EPT
NGRP16 = EPT16 // G
E_PAD = NW * EPT
N_ACC = 10240
ROWS_PT = N_ACC // NS
SINK = N_NODES
TBL_N = N_NODES + 16
TBL_W = TBL_N * 4
OUT2_R = EPT * 4 // 128


def _sc_mesh():
    return plsc.VectorSubcoreMesh(core_axis_name="c", subcore_axis_name="s")


def _agg_body(xlo_hbm, xhi_hbm, src16_hbm, dst16_hbm, ea_hbm,
              outx_hbm, oute_hbm,
              srcA, dstA, srcB, dstB, rowsA, rowsB, eav, ebuf, accx, acce,
              semA, semB):
    c = lax.axis_index("c")
    s = lax.axis_index("s")
    w = s * NC + c
    base = s * ROWS_PT

    def zero_body(i, _):
        zf = jnp.zeros((16,), jnp.float32)
        for t in range(D_HALF // 16):
            rowsA[i, pl.ds(t * 16, 16)] = zf
        eav[i, pl.ds(0, 16)] = zf
        return 0
    lax.fori_loop(0, G, zero_body, 0)
    for t in range(ROWS_PT // G):
        pltpu.sync_copy(rowsA, accx.at[pl.ds(base + t * G, G)])
        pltpu.sync_copy(eav, acce.at[pl.ds(base + t * G, G)])
    plsc.subcore_barrier()
    iota = lax.iota(jnp.int32, 16)

    def ea_body(j, _):
        pltpu.sync_copy(dst16_hbm.at[s, c * NGRP + j], dstA)
        pltpu.sync_copy(ea_hbm.at[w, pl.ds(j * G, G)], ebuf)

        def fill(k, _):
            q = k * 16 + iota
            v = plsc.load_gather(ebuf, [q >> 2, q & 3])
            plsc.store_scatter(eav, [q >> 2, q & 3], v)
            return 0
        lax.fori_loop(0, G * D_EA // 16, fill, 0)
        pltpu.sync_copy(eav, acce.at[dstA], add=True)
        return 0
    lax.fori_loop(0, NGRP, ea_body, 0)

    def make_x_loop(x_half):
        def start(idxbuf, rows, sem, j):
            pltpu.sync_copy(src16_hbm.at[s, j], idxbuf)
            pltpu.async_copy(x_half.at[idxbuf], rows, sem)

        def finish(idxbuf, dstbuf, rows, sem, j):
            pltpu.sync_copy(dst16_hbm.at[s, j], dstbuf)
            pltpu.make_async_copy(x_half.at[idxbuf], rows, sem).wait()
            pltpu.sync_copy(rows, accx.at[dstbuf], add=True)

        start(srcA, rowsA, semA, 0)

        def body(g, _):
            start(srcB, rowsB, semB, g + 1)
            finish(srcA, dstA, rowsA, semA, g)

            @pl.when(g + 2 < NGRP16)
            def _():
                start(srcA, rowsA, semA, g + 2)
            finish(srcB, dstB, rowsB, semB, g + 1)
            return 0
        lax.fori_loop(0, NGRP16 // 2, lambda g, u: body(g * 2, u), 0)

    @pl.when(c == 0)
    def _():
        make_x_loop(xlo_hbm)

    @pl.when(c == 1)
    def _():
        make_x_loop(xhi_hbm)

    plsc.subcore_barrier()
    pltpu.sync_copy(accx.at[pl.ds(base, ROWS_PT)],
                    outx_hbm.at[c, pl.ds(base, ROWS_PT)])
    pltpu.sync_copy(acce.at[pl.ds(base, ROWS_PT)],
                    oute_hbm.at[c, pl.ds(base, ROWS_PT)])


_sc_aggregate = functools.partial(
    pl.kernel,
    out_type=(
        jax.ShapeDtypeStruct((NC, N_ACC, D_HALF), jnp.float32),
        jax.ShapeDtypeStruct((NC, N_ACC, EA_PAD), jnp.float32),
    ),
    mesh=_sc_mesh(),
    scratch_types=[
        pltpu.VMEM((G,), jnp.int32),
        pltpu.VMEM((G,), jnp.int32),
        pltpu.VMEM((G,), jnp.int32),
        pltpu.VMEM((G,), jnp.int32),
        pltpu.VMEM((G, D_HALF), jnp.float32),
        pltpu.VMEM((G, D_HALF), jnp.float32),
        pltpu.VMEM((G, EA_PAD), jnp.float32),
        pltpu.VMEM((G, D_EA), jnp.float32),
        pltpu.VMEM_SHARED((N_ACC, D_HALF), jnp.float32),
        pltpu.VMEM_SHARED((N_ACC, EA_PAD), jnp.float32),
        pltpu.SemaphoreType.DMA,
        pltpu.SemaphoreType.DMA,
    ],
    compiler_params=pltpu.CompilerParams(
        use_tc_tiling_on_sc=False, needs_layout_passes=False),
)(_agg_body)


def _nea_body(tbl_hbm, src_hbm, dst_hbm, ox_hbm, oy_hbm, oz_hbm,
              srcv, dstv, tbl, ovx, ovy, ovz, sem):
    c = lax.axis_index("c")
    s = lax.axis_index("s")
    w = s * NC + c
    pltpu.sync_copy(tbl_hbm, tbl)
    pltpu.sync_copy(src_hbm.at[w], srcv)
    pltpu.sync_copy(dst_hbm.at[w], dstv)
    iota = lax.iota(jnp.int32, 16)

    def body(k, _):
        j = k // 8
        off = (k % 8) * 16
        si = srcv[j, pl.ds(off, 16)] * 4
        di = dstv[j, pl.ds(off, 16)] * 4
        eidx = k * 16 + iota
        for c3, ov in ((0, ovx), (1, ovy), (2, ovz)):
            a = plsc.load_gather(tbl, [si + c3])
            b = plsc.load_gather(tbl, [di + c3])
            plsc.store_scatter(ov, [eidx], b - a)
        return 0
    lax.fori_loop(0, EPT // 16, body, 0)
    pltpu.sync_copy(ovx, ox_hbm.at[pl.ds(w * EPT, EPT)])
    pltpu.sync_copy(ovy, oy_hbm.at[pl.ds(w * EPT, EPT)])
    pltpu.sync_copy(ovz, oz_hbm.at[pl.ds(w * EPT, EPT)])


_sc_edge_attr = functools.partial(
    pl.kernel,
    out_type=(
        jax.ShapeDtypeStruct((E_PAD,), jnp.float32),
        jax.ShapeDtypeStruct((E_PAD,), jnp.float32),
        jax.ShapeDtypeStruct((E_PAD,), jnp.float32),
    ),
    mesh=_sc_mesh(),
    scratch_types=[
        pltpu.VMEM((NGRP, G), jnp.int32),
        pltpu.VMEM((NGRP, G), jnp.int32),
        pltpu.VMEM((TBL_W,), jnp.float32),
        pltpu.VMEM((EPT,), jnp.float32),
        pltpu.VMEM((EPT,), jnp.float32),
        pltpu.VMEM((EPT,), jnp.float32),
        pltpu.SemaphoreType.DMA,
    ],
    compiler_params=pltpu.CompilerParams(needs_layout_passes=False),
)(_nea_body)


BN = 2000


def _dense_body(px, pe, pos4, batch2, w1s, w1h, w2s, w2h, wb4, wg1, wg2,
                xnew_ref, np_ref, gmax_ref):
    i = pl.program_id(0)
    dot = functools.partial(
        lax.dot_general,
        dimension_numbers=(((1,), (0,)), ((), ())),
        precision=lax.Precision.HIGHEST,
        preferred_element_type=jnp.float32,
    )
    ax = jnp.concatenate([px[0], px[1]], axis=1)
    ae = pe[0] + pe[1]
    sph = dot(ax, w1s[...]) + dot(ae, w2s[...])
    h = dot(ax, w1h[...]) + dot(ae, w2h[...])
    delta = dot(sph, wb4[...])
    np_ref[...] = pos4[...] + delta
    xnew_ref[...] = dot(h, wg1[...]) - dot(delta, wg2[...])
    nrm = jnp.sqrt(jnp.sum(delta * delta, axis=1, keepdims=True))
    bid = lax.broadcasted_iota(jnp.int32, (1, B_GRAPHS), 1)
    vals = jnp.where(batch2[...] == bid, nrm, -jnp.inf)
    bmax = jnp.max(vals, axis=0, keepdims=True)

    @pl.when(i == 0)
    def _():
        gmax_ref[...] = jnp.full((1, B_GRAPHS), -jnp.inf, jnp.float32)
    gmax_ref[...] = jnp.maximum(gmax_ref[...], bmax)


_tc_dense = pl.pallas_call(
    _dense_body,
    grid=(N_NODES // BN,),
    in_specs=[
        pl.BlockSpec((NC, BN, D_HALF), lambda i: (0, i, 0)),
        pl.BlockSpec((NC, BN, EA_PAD), lambda i: (0, i, 0)),
        pl.BlockSpec((BN, 4), lambda i: (i, 0)),
        pl.BlockSpec((BN, 1), lambda i: (i, 0)),
        pl.BlockSpec((D_IN, SPH_DIM), lambda i: (0, 0)),
        pl.BlockSpec((D_IN, D_IN), lambda i: (0, 0)),
        pl.BlockSpec((EA_PAD, SPH_DIM), lambda i: (0, 0)),
        pl.BlockSpec((EA_PAD, D_IN), lambda i: (0, 0)),
        pl.BlockSpec((SPH_DIM, 4), lambda i: (0, 0)),
        pl.BlockSpec((D_IN, D_IN), lambda i: (0, 0)),
        pl.BlockSpec((4, D_IN), lambda i: (0, 0)),
    ],
    out_specs=(
        pl.BlockSpec((BN, D_IN), lambda i: (i, 0)),
        pl.BlockSpec((BN, 4), lambda i: (i, 0)),
        pl.BlockSpec((1, B_GRAPHS), lambda i: (0, 0)),
    ),
    out_shape=(
        jax.ShapeDtypeStruct((N_NODES, D_IN), jnp.float32),
        jax.ShapeDtypeStruct((N_NODES, 4), jnp.float32),
        jax.ShapeDtypeStruct((1, B_GRAPHS), jnp.float32),
    ),
)


def kernel(x, pos, edge_index, edge_attr, batch, W_conv, W_bloom, W_gather):
    src, dst = edge_index[0], edge_index[1]
    npad = E_PAD - N_EDGES
    src_p = jnp.concatenate([src, jnp.zeros((npad,), jnp.int32)])
    dst_p = jnp.concatenate([dst, jnp.full((npad,), SINK, jnp.int32)])
    ea_p = jnp.pad(edge_attr, ((0, npad), (0, 0)))

    xlo = x[:, :D_HALF]
    xhi = x[:, D_HALF:]

    px, pe = _sc_aggregate(
        xlo, xhi,
        src_p.reshape(NS, NGRP16, G),
        dst_p.reshape(NS, NGRP16, G),
        ea_p.reshape(NW, EPT, D_EA),
    )

    w1 = W_conv[:D_IN]
    w2 = jnp.pad(W_conv[D_IN:], ((0, EA_PAD - D_EA), (0, 0)))
    wb_red = W_bloom.reshape(SPH_DIM, K_BLOOM, 3).mean(axis=1)
    wb4 = jnp.pad(wb_red, ((0, 0), (0, 1)))
    wg = K_BLOOM * W_gather
    wg1 = wg[:D_IN]
    wg2 = jnp.pad(wg[D_IN:], ((0, 1), (0, 0)))

    pos4 = jnp.pad(pos, ((0, 0), (0, 1)))
    x_new, np4, gmax = _tc_dense(
        px, pe, pos4, batch[:, None],
        w1[:, :SPH_DIM], w1[:, SPH_DIM:],
        w2[:, :SPH_DIM], w2[:, SPH_DIM:],
        wb4, wg1, wg2,
    )

    tbl = jnp.pad(np4, ((0, TBL_N - N_NODES), (0, 0)))
    nx, ny, nz = _sc_edge_attr(tbl.reshape(TBL_W),
                               src_p.reshape(NW, NGRP, G),
                               dst_p.reshape(NW, NGRP, G))
    new_edge_attr = jnp.stack(
        [nx[:N_EDGES], ny[:N_EDGES], nz[:N_EDGES]], axis=-1)
    new_pos = np4[:, :3]
    return (x_new, new_pos, edge_index, new_edge_attr, batch, gmax[0])


# speedup vs baseline: 1.6169x; 1.6169x over previous
import functools

import jax
import jax.numpy as jnp
from jax import lax
from jax.experimental import pallas as pl
from jax.experimental.pallas import tpu as pltpu
from jax.experimental.pallas import tpu_sc as plsc

N_NODES = 10000
N_EDGES = 320000
D_IN = 128
D_HALF = 64
SPH_DIM = 9
D_EA = 4
EA_PAD = 16
K_BLOOM = 4
B_GRAPHS = 8

NC = 2
NS = 16
NW = NC * NS
G = 128
EPT = 10240
NGRP = EPT // G
EPT16 = 2 * EPT
NGRP16 = EPT16 // G
E_PAD = NW * EPT
N_ACC = 10240
ROWS_PT = N_ACC // NS
SINK = N_NODES
TBL_N = N_NODES + 16
TBL_W = TBL_N * 4
OUT2_R = EPT * 4 // 128


def _sc_mesh():
    return plsc.VectorSubcoreMesh(core_axis_name="c", subcore_axis_name="s")


def _aggx_body(xlo_hbm, xhi_hbm, src16_hbm, dst16_hbm, outx_hbm,
               srcA, dstA, srcB, dstB, rowsA, rowsB, accx, semA, semB):
    c = lax.axis_index("c")
    s = lax.axis_index("s")
    base = s * ROWS_PT

    def zero_body(i, _):
        zf = jnp.zeros((16,), jnp.float32)
        for t in range(D_HALF // 16):
            rowsA[i, pl.ds(t * 16, 16)] = zf
        return 0
    lax.fori_loop(0, G, zero_body, 0)
    for t in range(ROWS_PT // G):
        pltpu.sync_copy(rowsA, accx.at[pl.ds(base + t * G, G)])
    plsc.subcore_barrier()

    def make_x_loop(x_half):
        def start(idxbuf, rows, sem, j):
            pltpu.sync_copy(src16_hbm.at[s, j], idxbuf)
            pltpu.async_copy(x_half.at[idxbuf], rows, sem)

        def finish(idxbuf, dstbuf, rows, sem, j):
            pltpu.sync_copy(dst16_hbm.at[s, j], dstbuf)
            pltpu.make_async_copy(x_half.at[idxbuf], rows, sem).wait()
            pltpu.sync_copy(rows, accx.at[dstbuf], add=True)

        start(srcA, rowsA, semA, 0)

        def body(g, _):
            start(srcB, rowsB, semB, g + 1)
            finish(srcA, dstA, rowsA, semA, g)

            @pl.when(g + 2 < NGRP16)
            def _():
                start(srcA, rowsA, semA, g + 2)
            finish(srcB, dstB, rowsB, semB, g + 1)
            return 0
        lax.fori_loop(0, NGRP16 // 2, lambda g, u: body(g * 2, u), 0)

    @pl.when(c == 0)
    def _():
        make_x_loop(xlo_hbm)

    @pl.when(c == 1)
    def _():
        make_x_loop(xhi_hbm)

    plsc.subcore_barrier()
    pltpu.sync_copy(accx.at[pl.ds(base, ROWS_PT)],
                    outx_hbm.at[c, pl.ds(base, ROWS_PT)])


_sc_agg_x = functools.partial(
    pl.kernel,
    out_type=jax.ShapeDtypeStruct((NC, N_ACC, D_HALF), jnp.float32),
    mesh=_sc_mesh(),
    scratch_types=[
        pltpu.VMEM((G,), jnp.int32),
        pltpu.VMEM((G,), jnp.int32),
        pltpu.VMEM((G,), jnp.int32),
        pltpu.VMEM((G,), jnp.int32),
        pltpu.VMEM((G, D_HALF), jnp.float32),
        pltpu.VMEM((G, D_HALF), jnp.float32),
        pltpu.VMEM_SHARED((N_ACC, D_HALF), jnp.float32),
        pltpu.SemaphoreType.DMA,
        pltpu.SemaphoreType.DMA,
    ],
    compiler_params=pltpu.CompilerParams(use_tc_tiling_on_sc=False),
)(_aggx_body)


def _agge_body(dst16_hbm, ea_hbm, oute_hbm, dstA, eav, ebuf, acce, sem):
    c = lax.axis_index("c")
    s = lax.axis_index("s")
    w = s * NC + c
    base = s * ROWS_PT

    def zero_body(i, _):
        eav[i, pl.ds(0, 16)] = jnp.zeros((16,), jnp.float32)
        return 0
    lax.fori_loop(0, G, zero_body, 0)
    for t in range(ROWS_PT // G):
        pltpu.sync_copy(eav, acce.at[pl.ds(base + t * G, G)])
    plsc.subcore_barrier()
    iota = lax.iota(jnp.int32, 16)

    pltpu.sync_copy(ea_hbm.at[w], ebuf)

    def ea_body(j, _):
        pltpu.sync_copy(dst16_hbm.at[s, c * NGRP + j], dstA)

        def fill(k, _):
            q = k * 16 + iota
            plsc.store_scatter(eav, [q >> 2, q & 3],
                               ebuf[pl.ds(j * G * D_EA + k * 16, 16)])
            return 0
        lax.fori_loop(0, G * D_EA // 16, fill, 0)
        pltpu.sync_copy(eav, acce.at[dstA], add=True)
        return 0
    lax.fori_loop(0, NGRP, ea_body, 0)

    plsc.subcore_barrier()
    pltpu.sync_copy(acce.at[pl.ds(base, ROWS_PT)],
                    oute_hbm.at[c, pl.ds(base, ROWS_PT)])


_sc_agg_e = functools.partial(
    pl.kernel,
    out_type=jax.ShapeDtypeStruct((NC, N_ACC, EA_PAD), jnp.float32),
    mesh=_sc_mesh(),
    scratch_types=[
        pltpu.VMEM((G,), jnp.int32),
        pltpu.VMEM((G, EA_PAD), jnp.float32),
        pltpu.VMEM((EPT * D_EA,), jnp.float32),
        pltpu.VMEM_SHARED((N_ACC, EA_PAD), jnp.float32),
        pltpu.SemaphoreType.DMA,
    ],
    compiler_params=pltpu.CompilerParams(
        use_tc_tiling_on_sc=False, needs_layout_passes=False),
)(_agge_body)


def _nea_body(tbl_hbm, src_hbm, dst_hbm, ox_hbm, oy_hbm, oz_hbm,
              srcv, dstv, tbl, ovx, ovy, ovz, sem):
    c = lax.axis_index("c")
    s = lax.axis_index("s")
    w = s * NC + c
    pltpu.sync_copy(tbl_hbm, tbl)
    pltpu.sync_copy(src_hbm.at[w], srcv)
    pltpu.sync_copy(dst_hbm.at[w], dstv)
    iota = lax.iota(jnp.int32, 16)

    def body(k, _):
        j = k // 8
        off = (k % 8) * 16
        si = srcv[j, pl.ds(off, 16)] * 4
        di = dstv[j, pl.ds(off, 16)] * 4
        eidx = k * 16 + iota
        for c3, ov in ((0, ovx), (1, ovy), (2, ovz)):
            a = plsc.load_gather(tbl, [si + c3])
            b = plsc.load_gather(tbl, [di + c3])
            plsc.store_scatter(ov, [eidx], b - a)
        return 0
    lax.fori_loop(0, EPT // 16, body, 0)
    pltpu.sync_copy(ovx, ox_hbm.at[pl.ds(w * EPT, EPT)])
    pltpu.sync_copy(ovy, oy_hbm.at[pl.ds(w * EPT, EPT)])
    pltpu.sync_copy(ovz, oz_hbm.at[pl.ds(w * EPT, EPT)])


_sc_edge_attr = functools.partial(
    pl.kernel,
    out_type=(
        jax.ShapeDtypeStruct((E_PAD,), jnp.float32),
        jax.ShapeDtypeStruct((E_PAD,), jnp.float32),
        jax.ShapeDtypeStruct((E_PAD,), jnp.float32),
    ),
    mesh=_sc_mesh(),
    scratch_types=[
        pltpu.VMEM((NGRP, G), jnp.int32),
        pltpu.VMEM((NGRP, G), jnp.int32),
        pltpu.VMEM((TBL_W,), jnp.float32),
        pltpu.VMEM((EPT,), jnp.float32),
        pltpu.VMEM((EPT,), jnp.float32),
        pltpu.VMEM((EPT,), jnp.float32),
        pltpu.SemaphoreType.DMA,
    ],
    compiler_params=pltpu.CompilerParams(needs_layout_passes=False),
)(_nea_body)


BN = 2000


def _dense_body(px, pe, pos4, batch2, w1s, w1h, w2s, w2h, wb4, wg1, wg2,
                xnew_ref, np_ref, gmax_ref):
    i = pl.program_id(0)
    dot = functools.partial(
        lax.dot_general,
        dimension_numbers=(((1,), (0,)), ((), ())),
        precision=lax.Precision.HIGHEST,
        preferred_element_type=jnp.float32,
    )
    ax = jnp.concatenate([px[0], px[1]], axis=1)
    ae = pe[0] + pe[1]
    sph = dot(ax, w1s[...]) + dot(ae, w2s[...])
    h = dot(ax, w1h[...]) + dot(ae, w2h[...])
    delta = dot(sph, wb4[...])
    np_ref[...] = pos4[...] + delta
    xnew_ref[...] = dot(h, wg1[...]) - dot(delta, wg2[...])
    nrm = jnp.sqrt(jnp.sum(delta * delta, axis=1, keepdims=True))
    bid = lax.broadcasted_iota(jnp.int32, (1, B_GRAPHS), 1)
    vals = jnp.where(batch2[...] == bid, nrm, -jnp.inf)
    bmax = jnp.max(vals, axis=0, keepdims=True)

    @pl.when(i == 0)
    def _():
        gmax_ref[...] = jnp.full((1, B_GRAPHS), -jnp.inf, jnp.float32)
    gmax_ref[...] = jnp.maximum(gmax_ref[...], bmax)


_tc_dense = pl.pallas_call(
    _dense_body,
    grid=(N_NODES // BN,),
    in_specs=[
        pl.BlockSpec((NC, BN, D_HALF), lambda i: (0, i, 0)),
        pl.BlockSpec((NC, BN, EA_PAD), lambda i: (0, i, 0)),
        pl.BlockSpec((BN, 4), lambda i: (i, 0)),
        pl.BlockSpec((BN, 1), lambda i: (i, 0)),
        pl.BlockSpec((D_IN, SPH_DIM), lambda i: (0, 0)),
        pl.BlockSpec((D_IN, D_IN), lambda i: (0, 0)),
        pl.BlockSpec((EA_PAD, SPH_DIM), lambda i: (0, 0)),
        pl.BlockSpec((EA_PAD, D_IN), lambda i: (0, 0)),
        pl.BlockSpec((SPH_DIM, 4), lambda i: (0, 0)),
        pl.BlockSpec((D_IN, D_IN), lambda i: (0, 0)),
        pl.BlockSpec((4, D_IN), lambda i: (0, 0)),
    ],
    out_specs=(
        pl.BlockSpec((BN, D_IN), lambda i: (i, 0)),
        pl.BlockSpec((BN, 4), lambda i: (i, 0)),
        pl.BlockSpec((1, B_GRAPHS), lambda i: (0, 0)),
    ),
    out_shape=(
        jax.ShapeDtypeStruct((N_NODES, D_IN), jnp.float32),
        jax.ShapeDtypeStruct((N_NODES, 4), jnp.float32),
        jax.ShapeDtypeStruct((1, B_GRAPHS), jnp.float32),
    ),
)


def kernel(x, pos, edge_index, edge_attr, batch, W_conv, W_bloom, W_gather):
    src, dst = edge_index[0], edge_index[1]
    npad = E_PAD - N_EDGES
    src_p = jnp.concatenate([src, jnp.zeros((npad,), jnp.int32)])
    dst_p = jnp.concatenate([dst, jnp.full((npad,), SINK, jnp.int32)])
    ea_flat = jnp.pad(edge_attr.reshape(N_EDGES * D_EA), (0, npad * D_EA))

    xlo = x[:, :D_HALF]
    xhi = x[:, D_HALF:]

    src16 = src_p.reshape(NS, NGRP16, G)
    dst16 = dst_p.reshape(NS, NGRP16, G)
    px = _sc_agg_x(xlo, xhi, src16, dst16)
    pe = _sc_agg_e(dst16, ea_flat.reshape(NW, EPT * D_EA))

    w1 = W_conv[:D_IN]
    w2 = jnp.pad(W_conv[D_IN:], ((0, EA_PAD - D_EA), (0, 0)))
    wb_red = W_bloom.reshape(SPH_DIM, K_BLOOM, 3).mean(axis=1)
    wb4 = jnp.pad(wb_red, ((0, 0), (0, 1)))
    wg = K_BLOOM * W_gather
    wg1 = wg[:D_IN]
    wg2 = jnp.pad(wg[D_IN:], ((0, 1), (0, 0)))

    pos4 = jnp.pad(pos, ((0, 0), (0, 1)))
    x_new, np4, gmax = _tc_dense(
        px, pe, pos4, batch[:, None],
        w1[:, :SPH_DIM], w1[:, SPH_DIM:],
        w2[:, :SPH_DIM], w2[:, SPH_DIM:],
        wb4, wg1, wg2,
    )

    tbl = jnp.pad(np4, ((0, TBL_N - N_NODES), (0, 0)))
    nx, ny, nz = _sc_edge_attr(tbl.reshape(TBL_W),
                               src_p.reshape(NW, NGRP, G),
                               dst_p.reshape(NW, NGRP, G))
    new_edge_attr = jnp.stack(
        [nx[:N_EDGES], ny[:N_EDGES], nz[:N_EDGES]], axis=-1)
    new_pos = np4[:, :3]
    return (x_new, new_pos, edge_index, new_edge_attr, batch, gmax[0])


# speedup vs baseline: 1.8786x; 1.1619x over previous
import functools

import jax
import jax.numpy as jnp
from jax import lax
from jax.experimental import pallas as pl
from jax.experimental.pallas import tpu as pltpu
from jax.experimental.pallas import tpu_sc as plsc

N_NODES = 10000
N_EDGES = 320000
D_IN = 128
D_HALF = 64
SPH_DIM = 9
D_EA = 4
EA_PAD = 16
K_BLOOM = 4
B_GRAPHS = 8

NC = 2
NS = 16
NW = NC * NS
G = 128
EPT = 10240
NGRP = EPT // G
EPT16 = 2 * EPT
NGRP16 = EPT16 // G
E_PAD = NW * EPT
N_ACC = 10240
ROWS_PT = N_ACC // NS
SINK = N_NODES
TBL_N = N_NODES + 16
TBL_W = TBL_N * 4
OUT2_R = EPT * 4 // 128


def _sc_mesh():
    return plsc.VectorSubcoreMesh(core_axis_name="c", subcore_axis_name="s")


def _aggx_body(xlo_hbm, xhi_hbm, src16_hbm, dst16_hbm, outx_hbm,
               srcA, dstA, srcB, dstB, rowsA, rowsB, srcS, dstS, accx,
               semA, semB):
    c = lax.axis_index("c")
    s = lax.axis_index("s")
    base = s * ROWS_PT

    def zero_body(i, _):
        zf = jnp.zeros((16,), jnp.float32)
        for t in range(D_HALF // 16):
            rowsA[i, pl.ds(t * 16, 16)] = zf
        return 0
    lax.fori_loop(0, G, zero_body, 0)
    for t in range(ROWS_PT // G):
        pltpu.sync_copy(rowsA, accx.at[pl.ds(base + t * G, G)])
    plsc.subcore_barrier()
    pltpu.sync_copy(src16_hbm.at[s], srcS)
    pltpu.sync_copy(dst16_hbm.at[s], dstS)

    def make_x_loop(x_half):
        def start(idxbuf, rows, sem, j):
            for t in range(G // 16):
                idxbuf[pl.ds(t * 16, 16)] = srcS[j, pl.ds(t * 16, 16)]
            pltpu.async_copy(x_half.at[idxbuf], rows, sem)

        def finish(idxbuf, dstbuf, rows, sem, j):
            for t in range(G // 16):
                dstbuf[pl.ds(t * 16, 16)] = dstS[j, pl.ds(t * 16, 16)]
            pltpu.make_async_copy(x_half.at[idxbuf], rows, sem).wait()
            pltpu.sync_copy(rows, accx.at[dstbuf], add=True)

        start(srcA, rowsA, semA, 0)

        def body(g, _):
            start(srcB, rowsB, semB, g + 1)
            finish(srcA, dstA, rowsA, semA, g)

            @pl.when(g + 2 < NGRP16)
            def _():
                start(srcA, rowsA, semA, g + 2)
            finish(srcB, dstB, rowsB, semB, g + 1)
            return 0
        lax.fori_loop(0, NGRP16 // 2, lambda g, u: body(g * 2, u), 0)

    @pl.when(c == 0)
    def _():
        make_x_loop(xlo_hbm)

    @pl.when(c == 1)
    def _():
        make_x_loop(xhi_hbm)

    plsc.subcore_barrier()
    pltpu.sync_copy(accx.at[pl.ds(base, ROWS_PT)],
                    outx_hbm.at[c, pl.ds(base, ROWS_PT)])


_sc_agg_x = functools.partial(
    pl.kernel,
    out_type=jax.ShapeDtypeStruct((NC, N_ACC, D_HALF), jnp.float32),
    mesh=_sc_mesh(),
    scratch_types=[
        pltpu.VMEM((G,), jnp.int32),
        pltpu.VMEM((G,), jnp.int32),
        pltpu.VMEM((G,), jnp.int32),
        pltpu.VMEM((G,), jnp.int32),
        pltpu.VMEM((G, D_HALF), jnp.float32),
        pltpu.VMEM((G, D_HALF), jnp.float32),
        pltpu.VMEM((NGRP16, G), jnp.int32),
        pltpu.VMEM((NGRP16, G), jnp.int32),
        pltpu.VMEM_SHARED((N_ACC, D_HALF), jnp.float32),
        pltpu.SemaphoreType.DMA,
        pltpu.SemaphoreType.DMA,
    ],
    compiler_params=pltpu.CompilerParams(use_tc_tiling_on_sc=False),
)(_aggx_body)


def _agge_body(dst16_hbm, ea_hbm, oute_hbm, dstA, eav, ebuf, dstS, acce, sem):
    c = lax.axis_index("c")
    s = lax.axis_index("s")
    w = s * NC + c
    base = s * ROWS_PT

    def zero_body(i, _):
        eav[i, pl.ds(0, 16)] = jnp.zeros((16,), jnp.float32)
        return 0
    lax.fori_loop(0, G, zero_body, 0)
    for t in range(ROWS_PT // G):
        pltpu.sync_copy(eav, acce.at[pl.ds(base + t * G, G)])
    plsc.subcore_barrier()
    iota = lax.iota(jnp.int32, 16)

    pltpu.sync_copy(ea_hbm.at[w], ebuf)
    pltpu.sync_copy(dst16_hbm.at[s, pl.ds(c * NGRP, NGRP)], dstS)

    def ea_body(j, _):
        for t in range(G // 16):
            dstA[pl.ds(t * 16, 16)] = dstS[j, pl.ds(t * 16, 16)]

        def fill(k, _):
            q = k * 16 + iota
            plsc.store_scatter(eav, [q >> 2, q & 3],
                               ebuf[pl.ds(j * G * D_EA + k * 16, 16)])
            return 0
        lax.fori_loop(0, G * D_EA // 16, fill, 0)
        pltpu.sync_copy(eav, acce.at[dstA], add=True)
        return 0
    lax.fori_loop(0, NGRP, ea_body, 0)

    plsc.subcore_barrier()
    pltpu.sync_copy(acce.at[pl.ds(base, ROWS_PT)],
                    oute_hbm.at[c, pl.ds(base, ROWS_PT)])


_sc_agg_e = functools.partial(
    pl.kernel,
    out_type=jax.ShapeDtypeStruct((NC, N_ACC, EA_PAD), jnp.float32),
    mesh=_sc_mesh(),
    scratch_types=[
        pltpu.VMEM((G,), jnp.int32),
        pltpu.VMEM((G, EA_PAD), jnp.float32),
        pltpu.VMEM((EPT * D_EA,), jnp.float32),
        pltpu.VMEM((NGRP, G), jnp.int32),
        pltpu.VMEM_SHARED((N_ACC, EA_PAD), jnp.float32),
        pltpu.SemaphoreType.DMA,
    ],
    compiler_params=pltpu.CompilerParams(
        use_tc_tiling_on_sc=False, needs_layout_passes=False),
)(_agge_body)


def _nea_body(tbl_hbm, src_hbm, dst_hbm, ox_hbm, oy_hbm, oz_hbm,
              srcv, dstv, tbl, ovx, ovy, ovz, sem):
    c = lax.axis_index("c")
    s = lax.axis_index("s")
    w = s * NC + c
    pltpu.sync_copy(tbl_hbm, tbl)
    pltpu.sync_copy(src_hbm.at[w], srcv)
    pltpu.sync_copy(dst_hbm.at[w], dstv)
    iota = lax.iota(jnp.int32, 16)

    def body(k, _):
        j = k // 8
        off = (k % 8) * 16
        si = srcv[j, pl.ds(off, 16)] * 4
        di = dstv[j, pl.ds(off, 16)] * 4
        eidx = k * 16 + iota
        for c3, ov in ((0, ovx), (1, ovy), (2, ovz)):
            a = plsc.load_gather(tbl, [si + c3])
            b = plsc.load_gather(tbl, [di + c3])
            plsc.store_scatter(ov, [eidx], b - a)
        return 0
    lax.fori_loop(0, EPT // 16, body, 0)
    pltpu.sync_copy(ovx, ox_hbm.at[pl.ds(w * EPT, EPT)])
    pltpu.sync_copy(ovy, oy_hbm.at[pl.ds(w * EPT, EPT)])
    pltpu.sync_copy(ovz, oz_hbm.at[pl.ds(w * EPT, EPT)])


_sc_edge_attr = functools.partial(
    pl.kernel,
    out_type=(
        jax.ShapeDtypeStruct((E_PAD,), jnp.float32),
        jax.ShapeDtypeStruct((E_PAD,), jnp.float32),
        jax.ShapeDtypeStruct((E_PAD,), jnp.float32),
    ),
    mesh=_sc_mesh(),
    scratch_types=[
        pltpu.VMEM((NGRP, G), jnp.int32),
        pltpu.VMEM((NGRP, G), jnp.int32),
        pltpu.VMEM((TBL_W,), jnp.float32),
        pltpu.VMEM((EPT,), jnp.float32),
        pltpu.VMEM((EPT,), jnp.float32),
        pltpu.VMEM((EPT,), jnp.float32),
        pltpu.SemaphoreType.DMA,
    ],
    compiler_params=pltpu.CompilerParams(needs_layout_passes=False),
)(_nea_body)


BN = 2000


def _dense_body(px, pe, pos4, batch2, w1s, w1h, w2s, w2h, wb4, wg1, wg2,
                xnew_ref, np_ref, gmax_ref):
    i = pl.program_id(0)
    dot = functools.partial(
        lax.dot_general,
        dimension_numbers=(((1,), (0,)), ((), ())),
        precision=lax.Precision.HIGHEST,
        preferred_element_type=jnp.float32,
    )
    ax = jnp.concatenate([px[0], px[1]], axis=1)
    ae = pe[0] + pe[1]
    sph = dot(ax, w1s[...]) + dot(ae, w2s[...])
    h = dot(ax, w1h[...]) + dot(ae, w2h[...])
    delta = dot(sph, wb4[...])
    np_ref[...] = pos4[...] + delta
    xnew_ref[...] = dot(h, wg1[...]) - dot(delta, wg2[...])
    nrm = jnp.sqrt(jnp.sum(delta * delta, axis=1, keepdims=True))
    bid = lax.broadcasted_iota(jnp.int32, (1, B_GRAPHS), 1)
    vals = jnp.where(batch2[...] == bid, nrm, -jnp.inf)
    bmax = jnp.max(vals, axis=0, keepdims=True)

    @pl.when(i == 0)
    def _():
        gmax_ref[...] = jnp.full((1, B_GRAPHS), -jnp.inf, jnp.float32)
    gmax_ref[...] = jnp.maximum(gmax_ref[...], bmax)


_tc_dense = pl.pallas_call(
    _dense_body,
    grid=(N_NODES // BN,),
    in_specs=[
        pl.BlockSpec((NC, BN, D_HALF), lambda i: (0, i, 0)),
        pl.BlockSpec((NC, BN, EA_PAD), lambda i: (0, i, 0)),
        pl.BlockSpec((BN, 4), lambda i: (i, 0)),
        pl.BlockSpec((BN, 1), lambda i: (i, 0)),
        pl.BlockSpec((D_IN, SPH_DIM), lambda i: (0, 0)),
        pl.BlockSpec((D_IN, D_IN), lambda i: (0, 0)),
        pl.BlockSpec((EA_PAD, SPH_DIM), lambda i: (0, 0)),
        pl.BlockSpec((EA_PAD, D_IN), lambda i: (0, 0)),
        pl.BlockSpec((SPH_DIM, 4), lambda i: (0, 0)),
        pl.BlockSpec((D_IN, D_IN), lambda i: (0, 0)),
        pl.BlockSpec((4, D_IN), lambda i: (0, 0)),
    ],
    out_specs=(
        pl.BlockSpec((BN, D_IN), lambda i: (i, 0)),
        pl.BlockSpec((BN, 4), lambda i: (i, 0)),
        pl.BlockSpec((1, B_GRAPHS), lambda i: (0, 0)),
    ),
    out_shape=(
        jax.ShapeDtypeStruct((N_NODES, D_IN), jnp.float32),
        jax.ShapeDtypeStruct((N_NODES, 4), jnp.float32),
        jax.ShapeDtypeStruct((1, B_GRAPHS), jnp.float32),
    ),
)


def kernel(x, pos, edge_index, edge_attr, batch, W_conv, W_bloom, W_gather):
    src, dst = edge_index[0], edge_index[1]
    npad = E_PAD - N_EDGES
    src_p = jnp.concatenate([src, jnp.zeros((npad,), jnp.int32)])
    dst_p = jnp.concatenate([dst, jnp.full((npad,), SINK, jnp.int32)])
    ea_flat = jnp.pad(edge_attr.reshape(N_EDGES * D_EA), (0, npad * D_EA))

    xlo = x[:, :D_HALF]
    xhi = x[:, D_HALF:]

    src16 = src_p.reshape(NS, NGRP16, G)
    dst16 = dst_p.reshape(NS, NGRP16, G)
    px = _sc_agg_x(xlo, xhi, src16, dst16)
    pe = _sc_agg_e(dst16, ea_flat.reshape(NW, EPT * D_EA))

    w1 = W_conv[:D_IN]
    w2 = jnp.pad(W_conv[D_IN:], ((0, EA_PAD - D_EA), (0, 0)))
    wb_red = W_bloom.reshape(SPH_DIM, K_BLOOM, 3).mean(axis=1)
    wb4 = jnp.pad(wb_red, ((0, 0), (0, 1)))
    wg = K_BLOOM * W_gather
    wg1 = wg[:D_IN]
    wg2 = jnp.pad(wg[D_IN:], ((0, 1), (0, 0)))

    pos4 = jnp.pad(pos, ((0, 0), (0, 1)))
    x_new, np4, gmax = _tc_dense(
        px, pe, pos4, batch[:, None],
        w1[:, :SPH_DIM], w1[:, SPH_DIM:],
        w2[:, :SPH_DIM], w2[:, SPH_DIM:],
        wb4, wg1, wg2,
    )

    tbl = jnp.pad(np4, ((0, TBL_N - N_NODES), (0, 0)))
    nx, ny, nz = _sc_edge_attr(tbl.reshape(TBL_W),
                               src_p.reshape(NW, NGRP, G),
                               dst_p.reshape(NW, NGRP, G))
    new_edge_attr = jnp.stack(
        [nx[:N_EDGES], ny[:N_EDGES], nz[:N_EDGES]], axis=-1)
    new_pos = np4[:, :3]
    return (x_new, new_pos, edge_index, new_edge_attr, batch, gmax[0])
